# Initial kernel scaffold; baseline (speedup 1.0000x reference)
import functools

import jax
import jax.numpy as jnp
from jax import lax
from jax.experimental import pallas as pl
from jax.experimental.pallas import tpu as pltpu
from jax.experimental.pallas import tpu_sc as plsc

N = 10000
E = 320000
F = 128
FE = 144
ALPHA = 0.2

NC = 2
NS = 16
NW = NC * NS
EPT = E // NW
CH = 80
NCHUNK = EPT // CH
RPT = N // NS

BLK = 400


def _phase1_body(x_ref, w_ref, a_ref, hext_ref, hlr_ref):
    h = jnp.dot(x_ref[...], w_ref[...], preferred_element_type=jnp.float32)
    lane = lax.broadcasted_iota(jnp.int32, (BLK, FE - F), 1)
    pad = jnp.where(lane == 0, 1.0, 0.0).astype(jnp.float32)
    hext_ref[...] = jnp.concatenate([h, pad], axis=1)
    hlr_ref[...] = jnp.dot(h, a_ref[...], preferred_element_type=jnp.float32)


def _phase1(x, W, A):
    return pl.pallas_call(
        _phase1_body,
        grid=(N // BLK,),
        in_specs=[
            pl.BlockSpec((BLK, F), lambda i: (i, 0)),
            pl.BlockSpec((F, F), lambda i: (0, 0)),
            pl.BlockSpec((F, 8), lambda i: (0, 0)),
        ],
        out_specs=[
            pl.BlockSpec((BLK, FE), lambda i: (i, 0)),
            pl.BlockSpec((BLK, 8), lambda i: (i, 0)),
        ],
        out_shape=[
            jax.ShapeDtypeStruct((N, FE), jnp.float32),
            jax.ShapeDtypeStruct((N, 8), jnp.float32),
        ],
    )(x, W, A)


_sc_mesh = plsc.VectorSubcoreMesh(core_axis_name="c", subcore_axis_name="s")


@functools.partial(
    pl.kernel,
    mesh=_sc_mesh,
    out_type=jax.ShapeDtypeStruct((NC * N, FE), jnp.float32),
    scratch_types=[
        pltpu.VMEM((N,), jnp.float32),
        pltpu.VMEM((N,), jnp.float32),
        pltpu.VMEM((CH,), jnp.int32),
        pltpu.VMEM((CH,), jnp.int32),
        pltpu.VMEM((CH,), jnp.float32),
        pltpu.VMEM((CH, FE), jnp.float32),
        pltpu.VMEM_SHARED((N, FE), jnp.float32),
        pltpu.SemaphoreType.DMA,
    ],
)
def _edge_kernel(hext, rowi, coli, hl, hr, zros, out,
                 hl_v, hr_v, row_c, col_c, w_c, rows_v, acc, sem):
    c = lax.axis_index("c")
    s = lax.axis_index("s")
    wid = c * NS + s
    pltpu.sync_copy(hl, hl_v)
    pltpu.sync_copy(hr, hr_v)
    pltpu.sync_copy(zros.at[pl.ds(s * RPT, RPT)], acc.at[pl.ds(s * RPT, RPT)])
    plsc.subcore_barrier()

    def chunk(k, carry):
        base = pl.multiple_of(wid * EPT + k * CH, CH)
        pltpu.sync_copy(rowi.at[pl.ds(base, CH)], row_c)
        pltpu.sync_copy(coli.at[pl.ds(base, CH)], col_c)
        gat = pltpu.async_copy(hext.at[col_c], rows_v, sem)
        for i in range(CH // 16):
            ridx = row_c[pl.ds(i * 16, 16)]
            cidx = col_c[pl.ds(i * 16, 16)]
            e = plsc.load_gather(hl_v, [ridx]) + plsc.load_gather(hr_v, [cidx])
            e = jnp.where(e > 0, e, ALPHA * e)
            w_c[pl.ds(i * 16, 16)] = jnp.exp(e)
        gat.wait()
        for i in range(CH // 16):
            wv = w_c[pl.ds(i * 16, 16)]
            for j in range(16):
                wspl = jnp.take(wv, jnp.full((16,), j, jnp.int32),
                                mode=lax.GatherScatterMode.PROMISE_IN_BOUNDS)
                eidx = i * 16 + j
                for r in range(FE // 16):
                    sl = pl.ds(r * 16, 16)
                    rows_v[eidx, sl] = rows_v[eidx, sl] * wspl
        pltpu.sync_copy(rows_v, acc.at[row_c], add=True)
        return carry

    lax.fori_loop(0, NCHUNK, chunk, 0)
    plsc.subcore_barrier()
    pltpu.sync_copy(acc.at[pl.ds(s * RPT, RPT)],
                    out.at[pl.ds(c * N + s * RPT, RPT)])


def _phase3_body(p_ref, o_ref):
    ss = p_ref[0] + p_ref[1]
    den = ss[:, F:F + 1] + 1e-16
    o_ref[...] = ss[:, :F] / den


def _phase3(partial):
    return pl.pallas_call(
        _phase3_body,
        grid=(N // BLK,),
        in_specs=[pl.BlockSpec((NC, BLK, FE), lambda i: (0, i, 0))],
        out_specs=pl.BlockSpec((BLK, F), lambda i: (i, 0)),
        out_shape=jax.ShapeDtypeStruct((N, F), jnp.float32),
    )(partial)


def kernel(x, edge_index, W, a_l, a_r):
    al = a_l.reshape(F)
    ar = a_r.reshape(F)
    A = jnp.zeros((F, 8), jnp.float32).at[:, 0].set(al).at[:, 1].set(ar)
    hext, hlr = _phase1(x, W, A)
    hl = hlr[:, 0]
    hr = hlr[:, 1]
    row = edge_index[0]
    col = edge_index[1]
    zeros = jnp.zeros((N, FE), jnp.float32)
    partial = _edge_kernel(hext, row, col, hl, hr, zeros)
    return _phase3(partial.reshape(NC, N, FE))


# speedup vs baseline: 30.3239x; 30.3239x over previous
import functools

import jax
import jax.numpy as jnp
from jax import lax
from jax.experimental import pallas as pl
from jax.experimental.pallas import tpu as pltpu
from jax.experimental.pallas import tpu_sc as plsc

N = 10000
E = 320000
F = 128
ALPHA = 0.2

NC = 2
NS = 16
NW = NC * NS
EPT = E // NW
CH = 80
NCHUNK = EPT // CH
NP = 10240
RPT = NP // NS

BLK = 400


def _phase1_body(x_ref, w_ref, a_ref, h_ref, hlr_ref):
    h = jnp.dot(x_ref[...], w_ref[...], preferred_element_type=jnp.float32)
    h_ref[...] = h
    hlr_ref[...] = jnp.dot(h, a_ref[...], preferred_element_type=jnp.float32)


def _phase1(x, W, A):
    return pl.pallas_call(
        _phase1_body,
        grid=(N // BLK,),
        in_specs=[
            pl.BlockSpec((BLK, F), lambda i: (i, 0)),
            pl.BlockSpec((F, F), lambda i: (0, 0)),
            pl.BlockSpec((F, 8), lambda i: (0, 0)),
        ],
        out_specs=[
            pl.BlockSpec((BLK, F), lambda i: (i, 0)),
            pl.BlockSpec((BLK, 8), lambda i: (i, 0)),
        ],
        out_shape=[
            jax.ShapeDtypeStruct((N, F), jnp.float32),
            jax.ShapeDtypeStruct((N, 8), jnp.float32),
        ],
    )(x, W, A)


def _lane_bcast(vec, j):
    idx = jnp.full((16, 1), j, jnp.int32)
    dnums = lax.GatherDimensionNumbers(
        offset_dims=(), collapsed_slice_dims=(0,), start_index_map=(0,))
    return lax.gather(vec, idx, dnums, (1,),
                      mode=lax.GatherScatterMode.PROMISE_IN_BOUNDS)


_sc_mesh = plsc.VectorSubcoreMesh(core_axis_name="c", subcore_axis_name="s")


@functools.partial(
    pl.kernel,
    mesh=_sc_mesh,
    out_type=[
        jax.ShapeDtypeStruct((NC * NP, F), jnp.float32),
        jax.ShapeDtypeStruct((NW * NP,), jnp.float32),
    ],
    scratch_types=[
        pltpu.VMEM((N,), jnp.float32),
        pltpu.VMEM((N,), jnp.float32),
        pltpu.VMEM((NP,), jnp.float32),
        pltpu.VMEM((CH,), jnp.int32),
        pltpu.VMEM((CH,), jnp.int32),
        pltpu.VMEM((CH,), jnp.float32),
        pltpu.VMEM((CH, F), jnp.float32),
        pltpu.VMEM_SHARED((NP, F), jnp.float32),
        pltpu.SemaphoreType.DMA,
    ],
    compiler_params=pltpu.CompilerParams(needs_layout_passes=False),
)
def _edge_kernel(h, rowi, coli, hl, hr, zros, zrosd, out, outd,
                 hl_v, hr_v, den_v, row_c, col_c, w_c, rows_v,
                 acc, sem):
    c = lax.axis_index("c")
    s = lax.axis_index("s")
    wid = c * NS + s
    pltpu.sync_copy(hl, hl_v)
    pltpu.sync_copy(hr, hr_v)
    pltpu.sync_copy(zrosd, den_v)
    pltpu.sync_copy(zros.at[pl.ds(s * RPT, RPT)], acc.at[pl.ds(s * RPT, RPT)])
    plsc.subcore_barrier()

    lane = lax.iota(jnp.int32, 16)

    def chunk(k, carry):
        base = pl.multiple_of(wid * EPT + k * CH, CH)
        pltpu.sync_copy(rowi.at[pl.ds(base, CH)], row_c)
        pltpu.sync_copy(coli.at[pl.ds(base, CH)], col_c)
        gat = pltpu.async_copy(h.at[col_c], rows_v, sem)
        for i in range(CH // 16):
            ridx = row_c[pl.ds(i * 16, 16)]
            cidx = col_c[pl.ds(i * 16, 16)]
            e = plsc.load_gather(hl_v, [ridx]) + plsc.load_gather(hr_v, [cidx])
            e = jnp.where(e > 0, e, ALPHA * e)
            w_c[pl.ds(i * 16, 16)] = jnp.exp(e)
        gat.wait()
        for i in range(CH // 16):
            wv = w_c[pl.ds(i * 16, 16)]
            rv = row_c[pl.ds(i * 16, 16)]
            for j in range(16):
                wspl = _lane_bcast(wv, j)
                plsc.addupdate_scatter(den_v, [rv], wv, mask=lane == j)
                eidx = i * 16 + j
                for r in range(F // 16):
                    sl = pl.ds(r * 16, 16)
                    rows_v[eidx, sl] = rows_v[eidx, sl] * wspl
        pltpu.sync_copy(rows_v, acc.at[row_c], add=True)
        return carry

    lax.fori_loop(0, NCHUNK, chunk, 0)
    pltpu.sync_copy(den_v, outd.at[pl.ds(wid * NP, NP)])
    plsc.subcore_barrier()
    pltpu.sync_copy(acc.at[pl.ds(s * RPT, RPT)],
                    out.at[pl.ds(c * NP + s * RPT, RPT)])


def _phase3_body(p_ref, d_ref, o_ref):
    ss = p_ref[0] + p_ref[1]
    den = jnp.sum(d_ref[...], axis=1) + 1e-16
    o_ref[...] = ss / den[:, None]


def _phase3(partial, denom):
    return pl.pallas_call(
        _phase3_body,
        grid=(N // BLK,),
        in_specs=[
            pl.BlockSpec((NC, BLK, F), lambda i: (0, i, 0)),
            pl.BlockSpec((BLK, NW), lambda i: (i, 0)),
        ],
        out_specs=pl.BlockSpec((BLK, F), lambda i: (i, 0)),
        out_shape=jax.ShapeDtypeStruct((N, F), jnp.float32),
    )(partial, denom)


def kernel(x, edge_index, W, a_l, a_r):
    al = a_l.reshape(F)
    ar = a_r.reshape(F)
    A = jnp.zeros((F, 8), jnp.float32).at[:, 0].set(al).at[:, 1].set(ar)
    h, hlr = _phase1(x, W, A)
    hl = hlr[:, 0]
    hr = hlr[:, 1]
    row = edge_index[0]
    col = edge_index[1]
    zeros = jnp.zeros((NP, F), jnp.float32)
    zerosd = jnp.zeros((NP,), jnp.float32)
    partial, denom = _edge_kernel(h, row, col, hl, hr, zeros, zerosd)
    return _phase3(partial.reshape(NC, NP, F), denom.reshape(NW, NP).T)


# speedup vs baseline: 57.0152x; 1.8802x over previous
import functools

import jax
import jax.numpy as jnp
from jax import lax
from jax.experimental import pallas as pl
from jax.experimental.pallas import tpu as pltpu
from jax.experimental.pallas import tpu_sc as plsc

N = 10000
E = 320000
F = 128
ALPHA = 0.2

NC = 2
NS = 16
NW = NC * NS
EPT = E // NW
CH = 80
NCHUNK = EPT // CH
NP = 10240
RPT = NP // NS

BLK = 400


def _phase1_body(x_ref, w_ref, a_ref, h_ref, hlr_ref):
    h = jnp.dot(x_ref[...], w_ref[...], preferred_element_type=jnp.float32)
    h_ref[...] = h
    hlr_ref[...] = jnp.dot(h, a_ref[...], preferred_element_type=jnp.float32)


def _phase1(x, W, A):
    return pl.pallas_call(
        _phase1_body,
        grid=(N // BLK,),
        in_specs=[
            pl.BlockSpec((BLK, F), lambda i: (i, 0)),
            pl.BlockSpec((F, F), lambda i: (0, 0)),
            pl.BlockSpec((F, 8), lambda i: (0, 0)),
        ],
        out_specs=[
            pl.BlockSpec((BLK, F), lambda i: (i, 0)),
            pl.BlockSpec((BLK, 8), lambda i: (i, 0)),
        ],
        out_shape=[
            jax.ShapeDtypeStruct((N, F), jnp.float32),
            jax.ShapeDtypeStruct((N, 8), jnp.float32),
        ],
    )(x, W, A)


def _lane_bcast(vec, j):
    idx = jnp.full((16, 1), j, jnp.int32)
    dnums = lax.GatherDimensionNumbers(
        offset_dims=(), collapsed_slice_dims=(0,), start_index_map=(0,))
    return lax.gather(vec, idx, dnums, (1,),
                      mode=lax.GatherScatterMode.PROMISE_IN_BOUNDS)


_sc_mesh = plsc.VectorSubcoreMesh(core_axis_name="c", subcore_axis_name="s")


@functools.partial(
    pl.kernel,
    mesh=_sc_mesh,
    out_type=[
        jax.ShapeDtypeStruct((E,), jnp.float32),
        jax.ShapeDtypeStruct((NW * NP,), jnp.float32),
    ],
    scratch_types=[
        pltpu.VMEM((N,), jnp.float32),
        pltpu.VMEM((N,), jnp.float32),
        pltpu.VMEM((NP,), jnp.float32),
        pltpu.VMEM((EPT,), jnp.int32),
        pltpu.VMEM((EPT,), jnp.int32),
        pltpu.VMEM((EPT,), jnp.float32),
    ],
    compiler_params=pltpu.CompilerParams(needs_layout_passes=False),
)
def _weight_kernel(rowi, coli, hl, hr, zrosd, w_out, dend,
                   hl_v, hr_v, den_v, row_t, col_t, w_all):
    c = lax.axis_index("c")
    s = lax.axis_index("s")
    wid = c * NS + s
    ebase = pl.multiple_of(wid * EPT, 16)
    pltpu.sync_copy(hl, hl_v)
    pltpu.sync_copy(hr, hr_v)
    pltpu.sync_copy(zrosd, den_v)
    pltpu.sync_copy(rowi.at[pl.ds(ebase, EPT)], row_t)
    pltpu.sync_copy(coli.at[pl.ds(ebase, EPT)], col_t)

    lane = lax.iota(jnp.int32, 16)

    def body(i, carry):
        sl = pl.ds(pl.multiple_of(i * 16, 16), 16)
        rv = row_t[sl]
        cv = col_t[sl]
        e = plsc.load_gather(hl_v, [rv]) + plsc.load_gather(hr_v, [cv])
        e = jnp.where(e > 0, e, ALPHA * e)
        w = jnp.exp(e)
        w_all[sl] = w
        for j in range(16):
            plsc.addupdate_scatter(den_v, [rv], w, mask=lane == j)
        return carry

    lax.fori_loop(0, EPT // 16, body, 0)
    pltpu.sync_copy(w_all, w_out.at[pl.ds(ebase, EPT)])
    pltpu.sync_copy(den_v, dend.at[pl.ds(pl.multiple_of(wid * NP, 16), NP)])


@functools.partial(
    pl.kernel,
    mesh=_sc_mesh,
    out_type=jax.ShapeDtypeStruct((NC * NP, F), jnp.float32),
    scratch_types=[
        pltpu.VMEM((3, CH, F), jnp.float32),
        pltpu.VMEM((2, CH), jnp.int32),
        pltpu.VMEM((2, CH), jnp.int32),
        pltpu.VMEM((2, CH), jnp.float32),
        pltpu.VMEM((2, CH), jnp.int32),
        pltpu.VMEM_SHARED((NP, F), jnp.float32),
        pltpu.SemaphoreType.DMA((3,)),
        pltpu.SemaphoreType.DMA((2,)),
        pltpu.SemaphoreType.DMA,
    ],
    compiler_params=pltpu.CompilerParams(needs_layout_passes=False),
)
def _spmm_kernel(h, rowi, coli, w_in, zros, out,
                 rows3, rc, ic, wc, row_s, acc, sem_g, sem_i, sem_s):
    c = lax.axis_index("c")
    s = lax.axis_index("s")
    wid = c * NS + s
    ebase = wid * EPT

    pltpu.sync_copy(zros.at[pl.ds(s * RPT, RPT)], acc.at[pl.ds(s * RPT, RPT)])
    plsc.subcore_barrier()

    def esl(k):
        return pl.ds(pl.multiple_of(ebase + k * CH, 16), CH)

    def idx_load_sync(k, b):
        pltpu.sync_copy(rowi.at[esl(k)], rc.at[b])
        pltpu.sync_copy(coli.at[esl(k)], ic.at[b])
        pltpu.sync_copy(w_in.at[esl(k)], wc.at[b])

    def idx_prefetch(k, b):
        pltpu.async_copy(rowi.at[esl(k)], rc.at[b], sem_i.at[b])
        pltpu.async_copy(coli.at[esl(k)], ic.at[b], sem_i.at[b])
        pltpu.async_copy(w_in.at[esl(k)], wc.at[b], sem_i.at[b])

    def idx_wait(k, b):
        pltpu.make_async_copy(rowi.at[esl(k)], rc.at[b], sem_i.at[b]).wait()
        pltpu.make_async_copy(coli.at[esl(k)], ic.at[b], sem_i.at[b]).wait()
        pltpu.make_async_copy(w_in.at[esl(k)], wc.at[b], sem_i.at[b]).wait()

    def gather_start(k, g, b):
        pltpu.async_copy(h.at[ic.at[b]], rows3.at[g], sem_g.at[g])

    def gather_wait(k, g, b):
        pltpu.make_async_copy(h.at[ic.at[b]], rows3.at[g], sem_g.at[g]).wait()

    def scatter_start(g, b):
        pltpu.async_copy(rows3.at[g], acc.at[row_s.at[b]], sem_s, add=True)

    def scatter_wait(g, b):
        pltpu.make_async_copy(rows3.at[g], acc.at[row_s.at[b]], sem_s).wait()

    def scale_and_stage(g, b):
        for i in range(CH // 16):
            sl = pl.ds(i * 16, 16)
            wv = wc[b, sl]
            row_s[b, sl] = rc[b, sl]
            for j in range(16):
                wspl = _lane_bcast(wv, j)
                eidx = i * 16 + j
                for r in range(F // 16):
                    fsl = pl.ds(r * 16, 16)
                    rows3[g, eidx, fsl] = rows3[g, eidx, fsl] * wspl

    idx_load_sync(0, 0)
    idx_load_sync(1, 1)
    gather_start(0, 0, 0)
    gather_start(1, 1, 1)
    gather_wait(0, 0, 0)
    scale_and_stage(0, 0)
    scatter_start(0, 0)
    idx_prefetch(2, 0)
    idx_wait(2, 0)
    gather_start(2, 2, 0)
    gather_wait(1, 1, 1)
    scale_and_stage(1, 1)
    scatter_start(1, 1)
    idx_prefetch(3, 1)

    def step(k, carry):
        b = lax.rem(k, 2)
        g = lax.rem(k, 3)
        bn = lax.rem(k + 1, 2)
        gn = lax.rem(k + 1, 3)
        scatter_wait(gn, b)

        @pl.when(k + 1 < NCHUNK)
        def _():
            idx_wait(k + 1, bn)
            gather_start(k + 1, gn, bn)

        gather_wait(k, g, b)
        scale_and_stage(g, b)
        scatter_start(g, b)

        @pl.when(k + 2 < NCHUNK)
        def _():
            idx_prefetch(k + 2, b)

        return carry

    lax.fori_loop(2, NCHUNK, step, 0)
    scatter_wait((NCHUNK - 2) % 3, (NCHUNK - 2) % 2)
    scatter_wait((NCHUNK - 1) % 3, (NCHUNK - 1) % 2)

    plsc.subcore_barrier()
    pltpu.sync_copy(acc.at[pl.ds(s * RPT, RPT)],
                    out.at[pl.ds(c * NP + s * RPT, RPT)])


def _phase3_body(p_ref, d_ref, o_ref):
    ss = p_ref[0] + p_ref[1]
    den = jnp.sum(d_ref[...], axis=1) + 1e-16
    o_ref[...] = ss / den[:, None]


def _phase3(partial, denom):
    return pl.pallas_call(
        _phase3_body,
        grid=(N // BLK,),
        in_specs=[
            pl.BlockSpec((NC, BLK, F), lambda i: (0, i, 0)),
            pl.BlockSpec((BLK, NW), lambda i: (i, 0)),
        ],
        out_specs=pl.BlockSpec((BLK, F), lambda i: (i, 0)),
        out_shape=jax.ShapeDtypeStruct((N, F), jnp.float32),
    )(partial, denom)


def kernel(x, edge_index, W, a_l, a_r):
    al = a_l.reshape(F)
    ar = a_r.reshape(F)
    A = jnp.zeros((F, 8), jnp.float32).at[:, 0].set(al).at[:, 1].set(ar)
    h, hlr = _phase1(x, W, A)
    hl = hlr[:, 0]
    hr = hlr[:, 1]
    row = edge_index[0]
    col = edge_index[1]
    zeros = jnp.zeros((NP, F), jnp.float32)
    zerosd = jnp.zeros((NP,), jnp.float32)
    w_e, denom = _weight_kernel(row, col, hl, hr, zerosd)
    partial = _spmm_kernel(h, row, col, w_e, zeros)
    return _phase3(partial.reshape(NC, NP, F), denom.reshape(NW, NP).T)
